# Initial kernel scaffold; baseline (speedup 1.0000x reference)
import jax
import jax.numpy as jnp
from jax.experimental import pallas as pl


def kernel(x, task_ids, leg_W, leg_b, trunc_W, trunc_b):
    raise NotImplementedError("write your pallas kernel here")


# speedup vs baseline: 2.4295x; 2.4295x over previous
import functools

import jax
import jax.numpy as jnp
from jax.experimental import pallas as pl
from jax.experimental.pallas import tpu as pltpu

_INTERPRET = False


def _body(t_ref, x_ref, w_ref, b_ref, tw_ref, tb_ref, out_ref, acc_ref):
    e = pl.program_id(0)
    num_e = pl.num_programs(0)
    mask = (t_ref[:] == e).astype(jnp.float32)
    y = jnp.dot(x_ref[:], w_ref[0], preferred_element_type=jnp.float32)
    y = y + b_ref[0]

    @pl.when(e == 0)
    def _():
        acc_ref[:] = mask * y

    @pl.when(e > 0)
    def _():
        acc_ref[:] = acc_ref[:] + mask * y

    @pl.when(e == num_e - 1)
    def _():
        out_ref[:] = (
            jnp.dot(acc_ref[:], tw_ref[:], preferred_element_type=jnp.float32)
            + tb_ref[:]
        )


def kernel(x, task_ids, leg_W, leg_b, trunc_W, trunc_b):
    n, d_in = x.shape
    num_e, _, d_tr = leg_W.shape
    d_out = trunc_W.shape[1]
    t2 = task_ids.astype(jnp.int32).reshape(n, 1)
    lb3 = leg_b.reshape(num_e, 1, d_tr)
    tb2 = trunc_b.reshape(1, d_out)

    return pl.pallas_call(
        _body,
        grid=(num_e,),
        in_specs=[
            pl.BlockSpec((n, 1), lambda e: (0, 0)),
            pl.BlockSpec((n, d_in), lambda e: (0, 0)),
            pl.BlockSpec((1, d_in, d_tr), lambda e: (e, 0, 0)),
            pl.BlockSpec((1, 1, d_tr), lambda e: (e, 0, 0)),
            pl.BlockSpec((d_tr, d_out), lambda e: (0, 0)),
            pl.BlockSpec((1, d_out), lambda e: (0, 0)),
        ],
        out_specs=pl.BlockSpec((n, d_out), lambda e: (0, 0)),
        out_shape=jax.ShapeDtypeStruct((n, d_out), jnp.float32),
        scratch_shapes=[pltpu.VMEM((n, d_tr), jnp.float32)],
        interpret=_INTERPRET,
    )(t2, x, leg_W, lb3, trunc_W, tb2)
